# baseline (speedup 1.0000x reference)
import jax
import jax.numpy as jnp
from jax import lax
from jax.experimental import pallas as pl
from jax.experimental.pallas import tpu as pltpu
from jax.experimental.pallas import tpu_sc as plsc

_B, _S, _D = 4, 8192, 1024
_SCALE = float(_D) ** 0.5
_N = _B * _S
_NC, _NS = 2, 16
_NW = _NC * _NS
_RPW = _N // _NW
_K = 16
_NSLOT = 2
_NCHUNK = _RPW // _K
_LANES = 16
_VPR = _D // _LANES


def _body(lat_hbm, ids_hbm, tab_hbm, out_hbm,
          ids_v, lat_v, tab_v, out_v, lat_sp, lat_sems, tab_sems, out_sems):
    wid = lax.axis_index("s") * _NC + lax.axis_index("c")
    sid = lax.axis_index("s")
    base = wid * _RPW
    pltpu.sync_copy(ids_hbm.at[pl.ds(base, _RPW)], ids_v)

    def start_in(c, b):
        r0 = base + c * _K
        pltpu.async_copy(lat_hbm.at[pl.ds(r0, _K)], lat_sp.at[sid, b],
                         lat_sems[b])
        pltpu.async_copy(tab_hbm.at[ids_v.at[pl.ds(c * _K, _K)]],
                         tab_v.at[b], tab_sems[b])

    def wait_in(c, b):
        r0 = base + c * _K
        pltpu.make_async_copy(
            lat_hbm.at[pl.ds(r0, _K)], lat_sp.at[sid, b], lat_sems[b]).wait()
        pltpu.make_async_copy(
            tab_hbm.at[ids_v.at[pl.ds(c * _K, _K)]],
            tab_v.at[b], tab_sems[b]).wait()

    def start_out(c, b):
        r0 = base + c * _K
        pltpu.async_copy(out_v.at[b], out_hbm.at[pl.ds(r0, _K)], out_sems[b])

    def wait_out(c, b):
        r0 = base + c * _K
        pltpu.make_async_copy(
            out_v.at[b], out_hbm.at[pl.ds(r0, _K)], out_sems[b]).wait()

    def compute(b):
        lat_b, tab_b, out_b = lat_v.at[b], tab_v.at[b], out_v.at[b]

        @plsc.parallel_loop(0, _K)
        def _(k):
            for j in range(_VPR):
                sl = pl.ds(j * _LANES, _LANES)
                out_b[k, sl] = lat_b[k, sl] * _SCALE + tab_b[k, sl]

    for b in range(_NSLOT):
        start_in(b, b)

    def step(gi, carry):
        g = gi * _NSLOT
        for b in range(_NSLOT):
            c = g + b
            wait_in(c, b)

            @pl.when(c >= _NSLOT)
            def _():
                wait_out(c - _NSLOT, b)

            pass
            start_out(c, b)

            @pl.when(c + _NSLOT < _NCHUNK)
            def _():
                start_in(c + _NSLOT, b)
        return carry

    lax.fori_loop(0, _NCHUNK // _NSLOT, step, 0)
    for b in range(_NSLOT):
        wait_out(_NCHUNK - _NSLOT + b, b)


_embed = pl.kernel(
    _body,
    out_type=jax.ShapeDtypeStruct((_N, _D), jnp.float32),
    mesh=plsc.VectorSubcoreMesh(core_axis_name="c", subcore_axis_name="s"),
    scratch_types=[
        pltpu.VMEM((_RPW,), jnp.int32),
        pltpu.VMEM((_NSLOT, _K, _D), jnp.float32),
        pltpu.VMEM((_NSLOT, _K, _D), jnp.float32),
        pltpu.VMEM((_NSLOT, _K, _D), jnp.float32),
        pltpu.VMEM_SHARED((_NS, _NSLOT, _K, _D), jnp.float32),
        [pltpu.SemaphoreType.DMA] * _NSLOT,
        [pltpu.SemaphoreType.DMA] * _NSLOT,
        [pltpu.SemaphoreType.DMA] * _NSLOT,
    ],
)


def _tc_body(lat_ref, out_ref):
    out_ref[...] = lat_ref[...] * 0.0


_tc_dummy = pl.pallas_call(
    _tc_body,
    out_shape=jax.ShapeDtypeStruct((_N, _D), jnp.float32),
    grid=(32,),
    in_specs=[pl.BlockSpec((_N // 32, _D), lambda i: (i, 0))],
    out_specs=pl.BlockSpec((_N // 32, _D), lambda i: (i, 0)),
)


@jax.jit
def kernel(latent_vectors, position_ids, position_table):
    lat = latent_vectors.reshape(_N, _D)
    ids = position_ids.reshape(_N)
    out = _embed(lat, ids, position_table)
    tc_out = _tc_dummy(lat)
    scalar = tc_out[0, 0]
    out = lax.cond(scalar > -1e30, lambda: out, lambda: tc_out)
    return out.reshape(_B, _S, _D)


# speedup vs baseline: 1.7395x; 1.7395x over previous
import jax
import jax.numpy as jnp
from jax import lax
from jax.experimental import pallas as pl
from jax.experimental.pallas import tpu as pltpu
from jax.experimental.pallas import tpu_sc as plsc

_B, _S, _D = 4, 8192, 1024
_SCALE = float(_D) ** 0.5
_N = _B * _S
_NC, _NS = 2, 16
_NW = _NC * _NS
_RPW = _N // _NW
_K = 16
_NCHUNK = _RPW // _K
_LANES = 16
_VPR = _D // _LANES
_LSLOT = 2
_TSLOT = 3
_OSLOT = 2


def _body(lat_hbm, ids_hbm, tab_hbm, out_hbm,
          ids_v, lat_v, tab_v, out_v, lat_sems, tab_sems, out_sems):
    wid = lax.axis_index("s") * _NC + lax.axis_index("c")
    base = wid * _RPW
    pltpu.sync_copy(ids_hbm.at[pl.ds(base, _RPW)], ids_v)

    def start_lat(c, b):
        r0 = base + c * _K
        pltpu.async_copy(lat_hbm.at[pl.ds(r0, _K)], lat_v.at[b], lat_sems[b])

    def wait_lat(c, b):
        r0 = base + c * _K
        pltpu.make_async_copy(
            lat_hbm.at[pl.ds(r0, _K)], lat_v.at[b], lat_sems[b]).wait()

    def start_tab(c, t):
        pltpu.async_copy(tab_hbm.at[ids_v.at[pl.ds(c * _K, _K)]],
                         tab_v.at[t], tab_sems[t])

    def wait_tab(c, t):
        pltpu.make_async_copy(
            tab_hbm.at[ids_v.at[pl.ds(c * _K, _K)]],
            tab_v.at[t], tab_sems[t]).wait()

    def start_out(c, b):
        r0 = base + c * _K
        pltpu.async_copy(out_v.at[b], out_hbm.at[pl.ds(r0, _K)], out_sems[b])

    def wait_out(c, b):
        r0 = base + c * _K
        pltpu.make_async_copy(
            out_v.at[b], out_hbm.at[pl.ds(r0, _K)], out_sems[b]).wait()

    def compute(lb, t, ob):
        lat_b, tab_b, out_b = lat_v.at[lb], tab_v.at[t], out_v.at[ob]

        @plsc.parallel_loop(0, _K)
        def _(k):
            for j in range(_VPR):
                sl = pl.ds(j * _LANES, _LANES)
                out_b[k, sl] = lat_b[k, sl] * _SCALE + tab_b[k, sl]

    start_lat(0, 0)
    start_tab(0, 0)
    start_lat(1, 1)
    start_tab(1, 1)

    def one_chunk(c, b, t):
        wait_lat(c, b)
        wait_tab(c, t)

        @pl.when(c + 2 < _NCHUNK)
        def _():
            start_tab(c + 2, (t + 2) % _TSLOT)

        @pl.when(c >= _OSLOT)
        def _():
            wait_out(c - _OSLOT, b)

        compute(b, t, b)
        start_out(c, b)

        @pl.when(c + _LSLOT < _NCHUNK)
        def _():
            start_lat(c + _LSLOT, b)

    def step(gi, carry):
        g = gi * 6
        for u in range(6):
            one_chunk(g + u, u % 2, u % 3)
        return carry

    lax.fori_loop(0, _NCHUNK // 6, step, 0)

    for c in range(_NCHUNK - _NCHUNK % 6, _NCHUNK):
        one_chunk(c, c % 2, c % 3)

    wait_out(_NCHUNK - 2, (_NCHUNK - 2) % 2)
    wait_out(_NCHUNK - 1, (_NCHUNK - 1) % 2)


_embed = pl.kernel(
    _body,
    out_type=jax.ShapeDtypeStruct((_N, _D), jnp.float32),
    mesh=plsc.VectorSubcoreMesh(core_axis_name="c", subcore_axis_name="s"),
    scratch_types=[
        pltpu.VMEM((_RPW,), jnp.int32),
        pltpu.VMEM((_LSLOT, _K, _D), jnp.float32),
        pltpu.VMEM((_TSLOT, _K, _D), jnp.float32),
        pltpu.VMEM((_OSLOT, _K, _D), jnp.float32),
        [pltpu.SemaphoreType.DMA] * _LSLOT,
        [pltpu.SemaphoreType.DMA] * _TSLOT,
        [pltpu.SemaphoreType.DMA] * _OSLOT,
    ],
)


@jax.jit
def kernel(latent_vectors, position_ids, position_table):
    lat = latent_vectors.reshape(_N, _D)
    ids = position_ids.reshape(_N)
    out = _embed(lat, ids, position_table)
    return out.reshape(_B, _S, _D)


# speedup vs baseline: 1.9543x; 1.1234x over previous
import jax
import jax.numpy as jnp
from jax import lax
from jax.experimental import pallas as pl
from jax.experimental.pallas import tpu as pltpu
from jax.experimental.pallas import tpu_sc as plsc

_B, _S, _D = 4, 8192, 1024
_SCALE = float(_D) ** 0.5
_N = _B * _S
_NC, _NS = 2, 16
_NW = _NC * _NS
_RPW = _N // _NW
_K = 16
_NSLOT = 2
_NCHUNK = _RPW // _K
_LANES = 16
_VPR = _D // _LANES


def _body(lat_hbm, ids_hbm, tab_hbm, out_hbm,
          ids_v, lat_v, tab_v, out_v, lat_sems, tab_sems, out_sems):
    wid = lax.axis_index("s") * _NC + lax.axis_index("c")
    base = wid * _RPW
    pltpu.sync_copy(ids_hbm.at[pl.ds(base, _RPW)], ids_v)

    def start_in(c, b):
        r0 = base + c * _K
        pltpu.async_copy(lat_hbm.at[pl.ds(r0, _K)], lat_v.at[b], lat_sems[b])
        pltpu.async_copy(tab_hbm.at[ids_v.at[pl.ds(c * _K, _K)]],
                         tab_v.at[b], tab_sems[b])

    def wait_in(c, b):
        r0 = base + c * _K
        pltpu.make_async_copy(
            lat_hbm.at[pl.ds(r0, _K)], lat_v.at[b], lat_sems[b]).wait()
        pltpu.make_async_copy(
            tab_hbm.at[ids_v.at[pl.ds(c * _K, _K)]],
            tab_v.at[b], tab_sems[b]).wait()

    def start_out(c, b):
        r0 = base + c * _K
        pltpu.async_copy(out_v.at[b], out_hbm.at[pl.ds(r0, _K)], out_sems[b])

    def wait_out(c, b):
        r0 = base + c * _K
        pltpu.make_async_copy(
            out_v.at[b], out_hbm.at[pl.ds(r0, _K)], out_sems[b]).wait()

    def compute(b):
        lat_b, tab_b, out_b = lat_v.at[b], tab_v.at[b], out_v.at[b]

        @plsc.parallel_loop(0, _K)
        def _(k):
            @plsc.parallel_loop(0, _VPR, unroll=8)
            def _(j):
                sl = pl.ds(j * _LANES, _LANES)
                out_b[k, sl] = lat_b[k, sl] * _SCALE + tab_b[k, sl]

    start_in(0, 0)
    start_in(1, 1)

    def step(gi, carry):
        g = gi * 2
        for b in range(2):
            c = g + b
            wait_in(c, b)

            @pl.when(c >= _NSLOT)
            def _():
                wait_out(c - _NSLOT, b)

            compute(b)
            start_out(c, b)

            @pl.when(c + _NSLOT < _NCHUNK)
            def _():
                start_in(c + _NSLOT, b)
        return carry

    lax.fori_loop(0, _NCHUNK // 2, step, 0)
    wait_out(_NCHUNK - 2, 0)
    wait_out(_NCHUNK - 1, 1)


_embed = pl.kernel(
    _body,
    out_type=jax.ShapeDtypeStruct((_N, _D), jnp.float32),
    mesh=plsc.VectorSubcoreMesh(core_axis_name="c", subcore_axis_name="s"),
    scratch_types=[
        pltpu.VMEM((_RPW,), jnp.int32),
        pltpu.VMEM((_NSLOT, _K, _D), jnp.float32),
        pltpu.VMEM((_NSLOT, _K, _D), jnp.float32),
        pltpu.VMEM((_NSLOT, _K, _D), jnp.float32),
        [pltpu.SemaphoreType.DMA] * _NSLOT,
        [pltpu.SemaphoreType.DMA] * _NSLOT,
        [pltpu.SemaphoreType.DMA] * _NSLOT,
    ],
)


@jax.jit
def kernel(latent_vectors, position_ids, position_table):
    lat = latent_vectors.reshape(_N, _D)
    ids = position_ids.reshape(_N)
    out = _embed(lat, ids, position_table)
    return out.reshape(_B, _S, _D)


# speedup vs baseline: 1.9682x; 1.0071x over previous
import jax
import jax.numpy as jnp
from jax import lax
from jax.experimental import pallas as pl
from jax.experimental.pallas import tpu as pltpu
from jax.experimental.pallas import tpu_sc as plsc

_B, _S, _D = 4, 8192, 1024
_SCALE = float(_D) ** 0.5
_N = _B * _S
_NC, _NS = 2, 16
_NW = _NC * _NS
_RPW = _N // _NW
_K = 16
_NSLOT = 2
_NCHUNK = _RPW // _K
_LANES = 16
_VPR = _D // _LANES


def _body(lat_hbm, ids_hbm, tab_hbm, out_hbm,
          ids_v, lat_v, tab_v, out_v, lat_sems, tab_sems, out_sems):
    wid = lax.axis_index("s") * _NC + lax.axis_index("c")
    base = wid * _RPW
    pltpu.sync_copy(ids_hbm.at[pl.ds(base, _RPW)], ids_v)

    def start_in(c, b):
        r0 = base + c * _K
        pltpu.async_copy(lat_hbm.at[pl.ds(r0, _K)], lat_v.at[b], lat_sems[b])
        pltpu.async_copy(tab_hbm.at[ids_v.at[pl.ds(c * _K, _K)]],
                         tab_v.at[b], tab_sems[b])

    def wait_in(c, b):
        r0 = base + c * _K
        pltpu.make_async_copy(
            lat_hbm.at[pl.ds(r0, _K)], lat_v.at[b], lat_sems[b]).wait()
        pltpu.make_async_copy(
            tab_hbm.at[ids_v.at[pl.ds(c * _K, _K)]],
            tab_v.at[b], tab_sems[b]).wait()

    def start_out(c, b):
        r0 = base + c * _K
        pltpu.async_copy(out_v.at[b], out_hbm.at[pl.ds(r0, _K)], out_sems[b])

    def wait_out(c, b):
        r0 = base + c * _K
        pltpu.make_async_copy(
            out_v.at[b], out_hbm.at[pl.ds(r0, _K)], out_sems[b]).wait()

    def compute(b):
        lat_b, tab_b, out_b = lat_v.at[b], tab_v.at[b], out_v.at[b]

        @plsc.parallel_loop(0, _K * _VPR, unroll=16)
        def _(i):
            k = i >> 6
            sl = pl.ds((i & (_VPR - 1)) * _LANES, _LANES)
            out_b[k, sl] = lat_b[k, sl] * _SCALE + tab_b[k, sl]

    start_in(0, 0)
    start_in(1, 1)

    def step(gi, carry):
        g = gi * 2
        for b in range(2):
            c = g + b
            wait_in(c, b)

            @pl.when(c >= _NSLOT)
            def _():
                wait_out(c - _NSLOT, b)

            compute(b)
            start_out(c, b)

            @pl.when(c + _NSLOT < _NCHUNK)
            def _():
                start_in(c + _NSLOT, b)
        return carry

    lax.fori_loop(0, _NCHUNK // 2, step, 0)
    wait_out(_NCHUNK - 2, 0)
    wait_out(_NCHUNK - 1, 1)


_embed = pl.kernel(
    _body,
    out_type=jax.ShapeDtypeStruct((_N, _D), jnp.float32),
    mesh=plsc.VectorSubcoreMesh(core_axis_name="c", subcore_axis_name="s"),
    scratch_types=[
        pltpu.VMEM((_RPW,), jnp.int32),
        pltpu.VMEM((_NSLOT, _K, _D), jnp.float32),
        pltpu.VMEM((_NSLOT, _K, _D), jnp.float32),
        pltpu.VMEM((_NSLOT, _K, _D), jnp.float32),
        [pltpu.SemaphoreType.DMA] * _NSLOT,
        [pltpu.SemaphoreType.DMA] * _NSLOT,
        [pltpu.SemaphoreType.DMA] * _NSLOT,
    ],
)


@jax.jit
def kernel(latent_vectors, position_ids, position_table):
    lat = latent_vectors.reshape(_N, _D)
    ids = position_ids.reshape(_N)
    out = _embed(lat, ids, position_table)
    return out.reshape(_B, _S, _D)
